# Initial kernel scaffold; baseline (speedup 1.0000x reference)
import jax
import jax.numpy as jnp
from jax.experimental import pallas as pl


def kernel(z, w12, b_eff, w_bd, b_bd):
    raise NotImplementedError("write your pallas kernel here")


# speedup vs baseline: 1.4589x; 1.4589x over previous
import jax
import jax.numpy as jnp
from jax.experimental import pallas as pl
from jax.experimental.pallas import tpu as pltpu

_PACK = 8


def _fused_kernel(z_ref, w_ref, b_ref, o_ref):
    acc = jnp.dot(z_ref[...], w_ref[...], preferred_element_type=jnp.float32)
    o_ref[...] = (acc + b_ref[...]).astype(o_ref.dtype)


def kernel(z, w12, b_eff, w_bd, b_bd):
    B, in_dim = z.shape
    out_dim = w12.shape[1]

    if B % _PACK != 0:
        zp, w, b = z, w12, b_eff.reshape(1, -1)
        rows, k, n = B, in_dim, out_dim
    else:
        zp = z.reshape(B // _PACK, _PACK * in_dim)
        w, b = w_bd, b_bd
        rows, k, n = B // _PACK, _PACK * in_dim, _PACK * out_dim

    tile = 4096
    if rows % tile != 0:
        tile = 8 * max(1, rows // (8 * 8))
    if rows <= tile:
        out = pl.pallas_call(
            _fused_kernel,
            out_shape=jax.ShapeDtypeStruct((rows, n), z.dtype),
        )(zp, w, b)
    else:
        steps = pl.cdiv(rows, tile)
        out = pl.pallas_call(
            _fused_kernel,
            out_shape=jax.ShapeDtypeStruct((rows, n), z.dtype),
            grid=(steps,),
            in_specs=[
                pl.BlockSpec((tile, k), lambda i: (i, 0)),
                pl.BlockSpec((k, n), lambda i: (0, 0)),
                pl.BlockSpec((1, n), lambda i: (0, 0)),
            ],
            out_specs=pl.BlockSpec((tile, n), lambda i: (i, 0)),
            compiler_params=pltpu.CompilerParams(
                dimension_semantics=("parallel",),
                vmem_limit_bytes=100 * 1024 * 1024,
            ),
        )(zp, w, b)

    return out.reshape(B, out_dim)


# speedup vs baseline: 1.7022x; 1.1668x over previous
import jax
import jax.numpy as jnp
from jax.experimental import pallas as pl
from jax.experimental.pallas import tpu as pltpu


def _direct_kernel(z_ref, w_ref, b_ref, o_ref):
    acc = jnp.dot(z_ref[...], w_ref[...], preferred_element_type=jnp.float32)
    o_ref[...] = (acc + b_ref[...]).astype(o_ref.dtype)


def kernel(z, w12, b_eff, w_bd, b_bd):
    B, in_dim = z.shape
    out_dim = w12.shape[1]
    b = b_eff.reshape(1, out_dim)

    tile = 8192
    if B % tile != 0:
        tile = 8 * max(1, B // (8 * 8))
    if B <= tile:
        out = pl.pallas_call(
            _direct_kernel,
            out_shape=jax.ShapeDtypeStruct((B, out_dim), z.dtype),
        )(z, w12, b)
    else:
        steps = pl.cdiv(B, tile)
        out = pl.pallas_call(
            _direct_kernel,
            out_shape=jax.ShapeDtypeStruct((B, out_dim), z.dtype),
            grid=(steps,),
            in_specs=[
                pl.BlockSpec((tile, in_dim), lambda i: (i, 0)),
                pl.BlockSpec((in_dim, out_dim), lambda i: (0, 0)),
                pl.BlockSpec((1, out_dim), lambda i: (0, 0)),
            ],
            out_specs=pl.BlockSpec((tile, out_dim), lambda i: (i, 0)),
            compiler_params=pltpu.CompilerParams(
                dimension_semantics=("parallel",),
                vmem_limit_bytes=100 * 1024 * 1024,
            ),
        )(z, w12, b)

    return out


# speedup vs baseline: 2.3711x; 1.3930x over previous
import jax
import jax.numpy as jnp
from jax.experimental import pallas as pl
from jax.experimental.pallas import tpu as pltpu


def _k3d(z_ref, w_ref, b_ref, o_ref):
    t = z_ref.shape[0]
    zb = z_ref[...].reshape(t * 8, z_ref.shape[2])
    acc = jnp.dot(zb, w_ref[...], preferred_element_type=jnp.float32)
    acc = acc + b_ref[...]
    o_ref[...] = acc.astype(o_ref.dtype).reshape(t, 8, o_ref.shape[2])


def kernel(z, w12, b_eff, w_bd, b_bd):
    B, in_dim = z.shape
    out_dim = w12.shape[1]
    b = b_eff.reshape(1, out_dim)

    zv = z.reshape(B // 8, 8, in_dim)
    rows = B // 8
    tile = 2048
    steps = pl.cdiv(rows, tile)
    out = pl.pallas_call(
        _k3d,
        out_shape=jax.ShapeDtypeStruct((rows, 8, out_dim), z.dtype),
        grid=(steps,),
        in_specs=[
            pl.BlockSpec((tile, 8, in_dim), lambda i: (i, 0, 0)),
            pl.BlockSpec((in_dim, out_dim), lambda i: (0, 0)),
            pl.BlockSpec((1, out_dim), lambda i: (0, 0)),
        ],
        out_specs=pl.BlockSpec((tile, 8, out_dim), lambda i: (i, 0, 0)),
        compiler_params=pltpu.CompilerParams(
            dimension_semantics=("parallel",),
            vmem_limit_bytes=60 * 1024 * 1024,
        ),
    )(zv, w12, b)

    return out.reshape(B, out_dim)
